# baseline (speedup 1.0000x reference)
import functools

import jax
import jax.numpy as jnp
from jax import lax
from jax.experimental import pallas as pl
from jax.experimental.pallas import tpu as pltpu
from jax.experimental.pallas import tpu_sc as plsc

NC = 2
NS = 16
NW = NC * NS
CHUNK = 128
GCH = 32
NBUF = 8
GLEAD = 6
SB = 16

_mesh = plsc.VectorSubcoreMesh(core_axis_name="c", subcore_axis_name="s")


def _deg_body(n_pad, k, dst_hbm, ew_hbm, degp_hbm, dstv, ewv, deg_sh):
    cid = lax.axis_index("c")
    sid = lax.axis_index("s")
    wid = cid * NS + sid
    rows_per_sub = n_pad // NS

    for i in range(CHUNK // 16):
        ewv[0, pl.ds(i * 16, 16)] = jnp.zeros((16,), jnp.float32)

    @pl.loop(0, rows_per_sub // CHUNK)
    def _zcopy(i):
        pltpu.sync_copy(
            ewv.at[0],
            deg_sh.at[pl.ds(sid * rows_per_sub + i * CHUNK, CHUNK)],
        )

    plsc.subcore_barrier()

    pltpu.sync_copy(dst_hbm.at[wid], dstv)
    pltpu.sync_copy(ew_hbm.at[wid], ewv)

    @pl.loop(0, k)
    def _scatter(j):
        pltpu.sync_copy(ewv.at[j], deg_sh.at[dstv.at[j]], add=True)

    plsc.subcore_barrier()

    pltpu.sync_copy(
        deg_sh.at[pl.ds(sid * rows_per_sub, rows_per_sub)],
        degp_hbm.at[cid, pl.ds(sid * rows_per_sub, rows_per_sub)],
    )


def _make_deg_kernel(n_pad, k):
    return pl.kernel(
        functools.partial(_deg_body, n_pad, k),
        out_type=jax.ShapeDtypeStruct((NC, n_pad), jnp.float32),
        mesh=_mesh,
        scratch_types=[
            pltpu.VMEM((k, CHUNK), jnp.int32),
            pltpu.VMEM((k, CHUNK), jnp.float32),
            pltpu.VMEM_SHARED((n_pad,), jnp.float32),
        ],
    )


def _layer_body(n_pad, k, d, y_hbm, src_hbm, dst_hbm, ew_hbm, accp_hbm,
                srcv, dstv, ewv, rows, acc_sh, gsem, ssem, csem):
    cid = lax.axis_index("c")
    sid = lax.axis_index("s")
    wid = cid * NS + sid
    rows_per_sub = n_pad // NS
    nq = d // 16
    nb = k // SB

    @pl.loop(0, GCH)
    def _zr(r):
        for q in range(nq):
            rows[0, r, pl.ds(q * 16, 16)] = jnp.zeros((16,), jnp.float32)

    @pl.loop(0, rows_per_sub // GCH)
    def _zcopy(i):
        pltpu.sync_copy(
            rows.at[0],
            acc_sh.at[pl.ds(sid * rows_per_sub + i * GCH, GCH)],
        )

    plsc.subcore_barrier()

    def stage_start(bi, sl):
        pltpu.async_copy(src_hbm.at[wid, pl.ds(bi * SB, SB)], srcv.at[sl], ssem)
        pltpu.async_copy(dst_hbm.at[wid, pl.ds(bi * SB, SB)], dstv.at[sl], ssem)
        pltpu.async_copy(ew_hbm.at[wid, pl.ds(bi * SB, SB)], ewv.at[sl], ssem)

    def stage_wait(bi, sl):
        pltpu.make_async_copy(src_hbm.at[wid, pl.ds(bi * SB, SB)], srcv.at[sl], ssem).wait()
        pltpu.make_async_copy(dst_hbm.at[wid, pl.ds(bi * SB, SB)], dstv.at[sl], ssem).wait()
        pltpu.make_async_copy(ew_hbm.at[wid, pl.ds(bi * SB, SB)], ewv.at[sl], ssem).wait()

    def start_gather(j, b):
        sl = (j // SB) % 2
        pltpu.async_copy(y_hbm.at[srcv.at[sl, j % SB]], rows.at[b], gsem)

    def wait_gather(j, b):
        sl = (j // SB) % 2
        pltpu.make_async_copy(y_hbm.at[srcv.at[sl, j % SB]], rows.at[b], gsem).wait()

    def start_scatter(j, b):
        sl = (j // SB) % 2
        pltpu.async_copy(rows.at[b], acc_sh.at[dstv.at[sl, j % SB]], csem, add=True)

    def wait_scatter(j, b):
        sl = (j // SB) % 2
        pltpu.make_async_copy(rows.at[b], acc_sh.at[dstv.at[sl, j % SB]], csem).wait()

    stage_start(0, 0)
    stage_wait(0, 0)

    for jp in range(GLEAD):
        start_gather(jp, jp)

    @pl.loop(0, k // NBUF)
    def _outer(j0):
        for b in range(NBUF):
            j = j0 * NBUF + b
            bi = j // SB
            jj = j % SB
            sl = bi % 2
            wait_gather(j, b)

            @pl.loop(0, GCH // 16)
            def _scale(g):
                ws = ewv[sl, jj, pl.ds(g * 16, 16)]
                for t in range(16):
                    s = ws[t]
                    for q in range(nq):
                        rows[b, g * 16 + t, pl.ds(q * 16, 16)] = (
                            rows[b, g * 16 + t, pl.ds(q * 16, 16)] * s)

            start_scatter(j, b)

            @pl.when(j >= 2)
            def _wsc():
                wait_scatter(j - 2, (j - 2) % NBUF)

            @pl.when(jnp.logical_and(jj == 1, bi + 1 < nb))
            def _rstage():
                stage_start(bi + 1, 1 - sl)

            @pl.when(jnp.logical_and(j + GLEAD < k, jj == SB - GLEAD))
            def _wstage():
                stage_wait(bi + 1, 1 - sl)

            @pl.when(j + GLEAD < k)
            def _next():
                start_gather(j + GLEAD, (j + GLEAD) % NBUF)

    wait_scatter(k - 2, (k - 2) % NBUF)
    wait_scatter(k - 1, (k - 1) % NBUF)

    plsc.subcore_barrier()

    pltpu.sync_copy(
        acc_sh.at[pl.ds(sid * rows_per_sub, rows_per_sub)],
        accp_hbm.at[cid, pl.ds(sid * rows_per_sub, rows_per_sub)],
    )


def _make_layer_kernel(n_pad, k, d):
    return pl.kernel(
        functools.partial(_layer_body, n_pad, k, d),
        out_type=jax.ShapeDtypeStruct((NC, n_pad, d), jnp.float32),
        mesh=_mesh,
        scratch_types=[
            pltpu.VMEM((2, SB, GCH), jnp.int32),
            pltpu.VMEM((2, SB, GCH), jnp.int32),
            pltpu.VMEM((2, SB, GCH), jnp.float32),
            pltpu.VMEM((NBUF, GCH, d), jnp.float32),
            pltpu.VMEM_SHARED((n_pad, d), jnp.float32),
            pltpu.SemaphoreType.DMA,
            pltpu.SemaphoreType.DMA,
            pltpu.SemaphoreType.DMA,
        ],
    )


def _tc1_body(x_ref, w_ref, degp_ref, y_ref, dinv_ref):
    deg = degp_ref[:, 0:1] + degp_ref[:, 1:2] + 1.0
    dinv = lax.rsqrt(deg)
    xw = jnp.dot(x_ref[...], w_ref[...], preferred_element_type=jnp.float32)
    y_ref[...] = xw * dinv
    dinv_ref[...] = dinv


def _tc2_body(accp_ref, y_ref, dinv_ref, b_ref, w_ref, y2_ref):
    acc = accp_ref[0] + accp_ref[1] + y_ref[...]
    h = jnp.maximum(dinv_ref[...] * acc + b_ref[...], 0.0)
    y2_ref[...] = jnp.dot(h, w_ref[...], preferred_element_type=jnp.float32) * dinv_ref[...]


def _tc3_body(accp_ref, y_ref, dinv_ref, b_ref, out_ref):
    acc = accp_ref[0] + accp_ref[1] + y_ref[...]
    out_ref[...] = dinv_ref[...] * acc + b_ref[...]


def kernel(x, edge_index, edge_weight, W1, b1, W2, b2):
    n, d = x.shape
    e = edge_weight.shape[0]
    o = W2.shape[1]

    n_pad = ((n + NS * CHUNK - 1) // (NS * CHUNK)) * (NS * CHUNK)
    step = NW * GCH * 2 * SB
    assert step % (NW * CHUNK) == 0
    e_pad = ((e + step - 1) // step) * step
    per_w = e_pad // NW
    k = per_w // GCH
    kd = per_w // CHUNK

    src = jnp.concatenate([edge_index[0], jnp.zeros((e_pad - e,), jnp.int32)])
    dst = jnp.concatenate([edge_index[1], jnp.zeros((e_pad - e,), jnp.int32)])
    ew = jnp.concatenate([edge_weight, jnp.zeros((e_pad - e,), jnp.float32)])
    src_w = src.reshape(NW, k, GCH)
    dst_w = dst.reshape(NW, k, GCH)
    ew_w = ew.reshape(NW, k, GCH)
    dst_wd = dst.reshape(NW, kd, CHUNK)
    ew_wd = ew.reshape(NW, kd, CHUNK)
    x_pad = jnp.concatenate([x, jnp.zeros((n_pad - n, d), x.dtype)], axis=0)

    deg_kernel = _make_deg_kernel(n_pad, kd)
    layer_kernel = _make_layer_kernel(n_pad, k, d)

    degp = deg_kernel(dst_wd, ew_wd)
    degp_t = degp.T

    y1, dinv = pl.pallas_call(
        _tc1_body,
        out_shape=[
            jax.ShapeDtypeStruct((n_pad, d), jnp.float32),
            jax.ShapeDtypeStruct((n_pad, 1), jnp.float32),
        ],
    )(x_pad, W1, degp_t)

    accp1 = layer_kernel(y1, src_w, dst_w, ew_w)

    y2 = pl.pallas_call(
        _tc2_body,
        out_shape=jax.ShapeDtypeStruct((n_pad, o), jnp.float32),
    )(accp1, y1, dinv, b1.reshape(1, -1), W2)

    accp2 = layer_kernel(y2, src_w, dst_w, ew_w)

    out = pl.pallas_call(
        _tc3_body,
        out_shape=jax.ShapeDtypeStruct((n_pad, o), jnp.float32),
    )(accp2, y2, dinv, b2.reshape(1, -1))

    return out[:n]


# speedup vs baseline: 1.1491x; 1.1491x over previous
import functools

import jax
import jax.numpy as jnp
from jax import lax
from jax.experimental import pallas as pl
from jax.experimental.pallas import tpu as pltpu
from jax.experimental.pallas import tpu_sc as plsc

NC = 2
NS = 16
NW = NC * NS
CHUNK = 128
GCH = 16
SB = 4

_mesh = plsc.VectorSubcoreMesh(core_axis_name="c", subcore_axis_name="s")


def _deg_body(n_pad, k, dst_hbm, ew_hbm, degp_hbm, dstv, ewv, deg_sh):
    cid = lax.axis_index("c")
    sid = lax.axis_index("s")
    wid = cid * NS + sid
    rows_per_sub = n_pad // NS

    for i in range(CHUNK // 16):
        ewv[0, pl.ds(i * 16, 16)] = jnp.zeros((16,), jnp.float32)

    @pl.loop(0, rows_per_sub // CHUNK)
    def _zcopy(i):
        pltpu.sync_copy(
            ewv.at[0],
            deg_sh.at[pl.ds(sid * rows_per_sub + i * CHUNK, CHUNK)],
        )

    plsc.subcore_barrier()

    pltpu.sync_copy(dst_hbm.at[wid], dstv)
    pltpu.sync_copy(ew_hbm.at[wid], ewv)

    @pl.loop(0, k)
    def _scatter(j):
        pltpu.sync_copy(ewv.at[j], deg_sh.at[dstv.at[j]], add=True)

    plsc.subcore_barrier()

    pltpu.sync_copy(
        deg_sh.at[pl.ds(sid * rows_per_sub, rows_per_sub)],
        degp_hbm.at[cid, pl.ds(sid * rows_per_sub, rows_per_sub)],
    )


def _make_deg_kernel(n_pad, k):
    return pl.kernel(
        functools.partial(_deg_body, n_pad, k),
        out_type=jax.ShapeDtypeStruct((NC, n_pad), jnp.float32),
        mesh=_mesh,
        scratch_types=[
            pltpu.VMEM((k, CHUNK), jnp.int32),
            pltpu.VMEM((k, CHUNK), jnp.float32),
            pltpu.VMEM_SHARED((n_pad,), jnp.float32),
        ],
    )


def _part_body(per_w, half, src_hbm, dst_hbm, ew_hbm,
               srcp_hbm, dstp_hbm, ewp_hbm, cnt_hbm,
               in_s, in_d, in_w, st_s, st_d0, st_d1, st_w, st_p0, st_p1,
               cntv, sh_s0, sh_d0, sh_w0, sh_s1, sh_d1, sh_w1):
    cid = lax.axis_index("c")
    sid = lax.axis_index("s")
    wid = cid * NS + sid
    base = sid * per_w
    trash = NS * per_w

    for i in range(CHUNK // 16):
        st_s[pl.ds(i * 16, 16)] = jnp.zeros((16,), jnp.int32)
        st_w[pl.ds(i * 16, 16)] = jnp.zeros((16,), jnp.float32)

    @pl.loop(0, per_w // CHUNK)
    def _z(i):
        for sh in (sh_s0, sh_d0, sh_s1, sh_d1):
            pltpu.sync_copy(st_s, sh.at[pl.ds(base + i * CHUNK, CHUNK)])
        for sh in (sh_w0, sh_w1):
            pltpu.sync_copy(st_w, sh.at[pl.ds(base + i * CHUNK, CHUNK)])

    pltpu.sync_copy(src_hbm.at[wid], in_s)
    pltpu.sync_copy(dst_hbm.at[wid], in_d)
    pltpu.sync_copy(ew_hbm.at[wid], in_w)

    iota16 = lax.iota(jnp.int32, 16)

    @pl.loop(0, per_w // CHUNK, init_carry=(jnp.int32(0), jnp.int32(0)))
    def _compact(ch, carry):
        off0c, off1c = carry

        @pl.loop(0, CHUNK // 16, init_carry=(off0c, off1c))
        def _grp(g, c2):
            off0, off1 = c2
            s16 = in_s[pl.ds(ch * CHUNK + g * 16, 16)]
            d16 = in_d[pl.ds(ch * CHUNK + g * 16, 16)]
            w16 = in_w[pl.ds(ch * CHUNK + g * 16, 16)]
            m0i = jnp.clip(half - d16, 0, 1)
            pos0 = jnp.full((16,), trash, jnp.int32)
            pos1 = jnp.full((16,), trash, jnp.int32)
            run0 = off0
            run1 = off1
            for t in range(16):
                mt = m0i[t]
                dt = iota16 - t
                oh = jnp.clip(1 - dt * dt, 0, 1)
                pos0 = pos0 + oh * mt * (base + run0 - trash)
                pos1 = pos1 + oh * (1 - mt) * (base + run1 - trash)
                run0 = run0 + mt
                run1 = run1 + (1 - mt)
            st_s[pl.ds(g * 16, 16)] = s16
            st_d0[pl.ds(g * 16, 16)] = d16
            st_d1[pl.ds(g * 16, 16)] = d16 - half
            st_w[pl.ds(g * 16, 16)] = w16
            st_p0[0, pl.ds(g * 16, 16)] = pos0
            st_p1[0, pl.ds(g * 16, 16)] = pos1
            return (run0, run1)

        off0n, off1n = _grp
        pltpu.sync_copy(st_s, sh_s0.at[st_p0.at[0]], add=True)
        pltpu.sync_copy(st_d0, sh_d0.at[st_p0.at[0]], add=True)
        pltpu.sync_copy(st_w, sh_w0.at[st_p0.at[0]], add=True)
        pltpu.sync_copy(st_s, sh_s1.at[st_p1.at[0]], add=True)
        pltpu.sync_copy(st_d1, sh_d1.at[st_p1.at[0]], add=True)
        pltpu.sync_copy(st_w, sh_w1.at[st_p1.at[0]], add=True)
        return (off0n, off1n)

    off0, off1 = _compact

    def padded(c):
        g2 = 2 * GCH
        sh = g2.bit_length() - 1
        return jnp.maximum(((c + g2 - 1) >> sh) << sh, g2)

    oh0 = jnp.clip(1 - iota16 * iota16, 0, 1)
    cntv[0, pl.ds(0, 16)] = oh0 * padded(off0)
    cntv[1, pl.ds(0, 16)] = oh0 * padded(off1)

    pltpu.sync_copy(cntv.at[0], cnt_hbm.at[0, wid])
    pltpu.sync_copy(cntv.at[1], cnt_hbm.at[1, wid])
    pltpu.sync_copy(sh_s0.at[pl.ds(base, per_w)], srcp_hbm.at[0, wid])
    pltpu.sync_copy(sh_d0.at[pl.ds(base, per_w)], dstp_hbm.at[0, wid])
    pltpu.sync_copy(sh_w0.at[pl.ds(base, per_w)], ewp_hbm.at[0, wid])
    pltpu.sync_copy(sh_s1.at[pl.ds(base, per_w)], srcp_hbm.at[1, wid])
    pltpu.sync_copy(sh_d1.at[pl.ds(base, per_w)], dstp_hbm.at[1, wid])
    pltpu.sync_copy(sh_w1.at[pl.ds(base, per_w)], ewp_hbm.at[1, wid])


def _make_part_kernel(per_w, half):
    return pl.kernel(
        functools.partial(_part_body, per_w, half),
        out_type=[
            jax.ShapeDtypeStruct((2, NW, per_w), jnp.int32),
            jax.ShapeDtypeStruct((2, NW, per_w), jnp.int32),
            jax.ShapeDtypeStruct((2, NW, per_w), jnp.float32),
            jax.ShapeDtypeStruct((2, NW, 16), jnp.int32),
        ],
        mesh=_mesh,
        scratch_types=[
            pltpu.VMEM((per_w,), jnp.int32),
            pltpu.VMEM((per_w,), jnp.int32),
            pltpu.VMEM((per_w,), jnp.float32),
            pltpu.VMEM((CHUNK,), jnp.int32),
            pltpu.VMEM((CHUNK,), jnp.int32),
            pltpu.VMEM((CHUNK,), jnp.int32),
            pltpu.VMEM((CHUNK,), jnp.float32),
            pltpu.VMEM((1, CHUNK), jnp.int32),
            pltpu.VMEM((1, CHUNK), jnp.int32),
            pltpu.VMEM((2, 16), jnp.int32),
            pltpu.VMEM_SHARED((NS * per_w + 16,), jnp.int32),
            pltpu.VMEM_SHARED((NS * per_w + 16,), jnp.int32),
            pltpu.VMEM_SHARED((NS * per_w + 16,), jnp.float32),
            pltpu.VMEM_SHARED((NS * per_w + 16,), jnp.int32),
            pltpu.VMEM_SHARED((NS * per_w + 16,), jnp.int32),
            pltpu.VMEM_SHARED((NS * per_w + 16,), jnp.float32),
        ],
    )


def _layer_body(n, n2, half, kmax, d, y_hbm, srcp_hbm, dstp_hbm, ewp_hbm,
                cnt_hbm, acc_hbm, srcv, dstv, ewv, rows, cntv, y_sh, acc_sh,
                gsem, ssem):
    cid = lax.axis_index("c")
    sid = lax.axis_index("s")
    nq = d // 16
    acc_rows_per_sub = half // NS
    y_full = n2 // NS
    y_last = n - (NS - 1) * y_full

    @pl.loop(0, GCH)
    def _zr(r):
        for q in range(nq):
            rows[0, r, pl.ds(q * 16, 16)] = jnp.zeros((16,), jnp.float32)

    @pl.loop(0, acc_rows_per_sub // GCH)
    def _zcopy(i):
        pltpu.sync_copy(
            rows.at[0],
            acc_sh.at[pl.ds(sid * acc_rows_per_sub + i * GCH, GCH)],
        )

    @pl.when(sid < NS - 1)
    def _ycopy_full():
        pltpu.sync_copy(
            y_hbm.at[pl.ds(sid * y_full, y_full)],
            y_sh.at[pl.ds(sid * y_full, y_full)],
        )

    @pl.when(sid == NS - 1)
    def _ycopy_last():
        pltpu.sync_copy(
            y_hbm.at[pl.ds((NS - 1) * y_full, y_last)],
            y_sh.at[pl.ds((NS - 1) * y_full, y_last)],
        )

    plsc.subcore_barrier()

    for seg in range(2):
        w = sid * 2 + seg

        pltpu.sync_copy(cnt_hbm.at[cid, w], cntv.at[0])
        kc = lax.shift_right_logical(cntv[0, pl.ds(0, 16)][0],
                                     GCH.bit_length() - 1)
        kc = jnp.clip(kc, 2, kmax)

        def stage_start(bi, sl):
            pltpu.async_copy(srcp_hbm.at[cid, w, pl.ds(bi * SB, SB)], srcv.at[sl], ssem)
            pltpu.async_copy(dstp_hbm.at[cid, w, pl.ds(bi * SB, SB)], dstv.at[sl], ssem)
            pltpu.async_copy(ewp_hbm.at[cid, w, pl.ds(bi * SB, SB)], ewv.at[sl], ssem)

        def stage_wait(bi, sl):
            pltpu.make_async_copy(srcp_hbm.at[cid, w, pl.ds(bi * SB, SB)], srcv.at[sl], ssem).wait()
            pltpu.make_async_copy(dstp_hbm.at[cid, w, pl.ds(bi * SB, SB)], dstv.at[sl], ssem).wait()
            pltpu.make_async_copy(ewp_hbm.at[cid, w, pl.ds(bi * SB, SB)], ewv.at[sl], ssem).wait()

        def start_gather(j, b):
            sl = (j // SB) % 2
            pltpu.async_copy(y_sh.at[srcv.at[sl, j % SB]], rows.at[b], gsem)

        def wait_gather(j, b):
            sl = (j // SB) % 2
            pltpu.make_async_copy(y_sh.at[srcv.at[sl, j % SB]], rows.at[b], gsem).wait()

        nb = lax.shift_right_logical(kc + SB - 1, SB.bit_length() - 1)

        stage_start(0, 0)
        stage_wait(0, 0)

        @pl.when(nb > 1)
        def _s1():
            stage_start(1, 1)

        start_gather(0, 0)
        start_gather(1, 1)

        @pl.loop(0, lax.shift_right_logical(kc, 1))
        def _outer(j0):
            for b in range(2):
                j = j0 * 2 + b
                bi = j // SB
                jj = j % SB
                sl = bi % 2
                wait_gather(j, b)

                @pl.loop(0, GCH // 16)
                def _scale(g):
                    ws = ewv[sl, jj, pl.ds(g * 16, 16)]
                    for t in range(16):
                        s = ws[t]
                        for q in range(nq):
                            rows[b, g * 16 + t, pl.ds(q * 16, 16)] = (
                                rows[b, g * 16 + t, pl.ds(q * 16, 16)] * s)

                pltpu.sync_copy(rows.at[b], acc_sh.at[dstv.at[sl, jj]], add=True)

                @pl.when(jnp.logical_and(j + 2 < kc, jj == SB - 2))
                def _wstage():
                    stage_wait(bi + 1, 1 - sl)

                @pl.when(j + 2 < kc)
                def _next():
                    start_gather(j + 2, b)

                @pl.when(jnp.logical_and(jj == SB - 1, bi + 2 < nb))
                def _rstage():
                    stage_start(bi + 2, sl)

    plsc.subcore_barrier()

    pltpu.sync_copy(
        acc_sh.at[pl.ds(sid * acc_rows_per_sub, acc_rows_per_sub)],
        acc_hbm.at[pl.ds(cid * half + sid * acc_rows_per_sub,
                         acc_rows_per_sub)],
    )


def _make_layer_kernel(n, n2, half, kmax, d):
    return pl.kernel(
        functools.partial(_layer_body, n, n2, half, kmax, d),
        out_type=jax.ShapeDtypeStruct((2 * half, d), jnp.float32),
        mesh=_mesh,
        scratch_types=[
            pltpu.VMEM((2, SB, GCH), jnp.int32),
            pltpu.VMEM((2, SB, GCH), jnp.int32),
            pltpu.VMEM((2, SB, GCH), jnp.float32),
            pltpu.VMEM((2, GCH, d), jnp.float32),
            pltpu.VMEM((1, 16), jnp.int32),
            pltpu.VMEM_SHARED((n, d), jnp.float32),
            pltpu.VMEM_SHARED((half, d), jnp.float32),
            pltpu.SemaphoreType.DMA,
            pltpu.SemaphoreType.DMA,
        ],
    )


def _tc1_body(x_ref, w_ref, degp_ref, y_ref, dinv_ref):
    deg = degp_ref[:, 0:1] + degp_ref[:, 1:2] + 1.0
    dinv = lax.rsqrt(deg)
    xw = jnp.dot(x_ref[...], w_ref[...], preferred_element_type=jnp.float32)
    y_ref[...] = xw * dinv
    dinv_ref[...] = dinv


def _tc2_body(acc_ref, y_ref, dinv_ref, b_ref, w_ref, y2_ref):
    acc = acc_ref[...] + y_ref[...]
    h = jnp.maximum(dinv_ref[...] * acc + b_ref[...], 0.0)
    y2_ref[...] = jnp.dot(h, w_ref[...], preferred_element_type=jnp.float32) * dinv_ref[...]


def _tc3_body(acc_ref, y_ref, dinv_ref, b_ref, out_ref):
    acc = acc_ref[...] + y_ref[...]
    out_ref[...] = dinv_ref[...] * acc + b_ref[...]


def kernel(x, edge_index, edge_weight, W1, b1, W2, b2):
    n, d = x.shape
    e = edge_weight.shape[0]
    o = W2.shape[1]

    n_pad = ((n + NS * CHUNK - 1) // (NS * CHUNK)) * (NS * CHUNK)
    half = n_pad // 2
    n2 = ((n + NS * 8 - 1) // (NS * 8)) * (NS * 8)
    step = NW * 128
    e_pad = ((e + step - 1) // step) * step
    per_w = e_pad // NW
    kd = per_w // CHUNK
    kmax = per_w // GCH

    src = jnp.concatenate([edge_index[0], jnp.zeros((e_pad - e,), jnp.int32)])
    dst = jnp.concatenate([edge_index[1], jnp.zeros((e_pad - e,), jnp.int32)])
    ew = jnp.concatenate([edge_weight, jnp.zeros((e_pad - e,), jnp.float32)])
    src_w = src.reshape(NW, per_w)
    dst_w = dst.reshape(NW, per_w)
    ew_w = ew.reshape(NW, per_w)
    dst_wd = dst.reshape(NW, kd, CHUNK)
    ew_wd = ew.reshape(NW, kd, CHUNK)
    x_pad = jnp.concatenate([x, jnp.zeros((n_pad - n, d), x.dtype)], axis=0)

    deg_kernel = _make_deg_kernel(n_pad, kd)
    part_kernel = _make_part_kernel(per_w, half)
    layer_kernel = _make_layer_kernel(n, n2, half, kmax, d)

    degp = deg_kernel(dst_wd, ew_wd)
    srcp, dstp, ewp, cnt = part_kernel(src_w, dst_w, ew_w)
    srcp4 = srcp.reshape(2, NW, kmax, GCH)
    dstp4 = dstp.reshape(2, NW, kmax, GCH)
    ewp4 = ewp.reshape(2, NW, kmax, GCH)

    y1, dinv = pl.pallas_call(
        _tc1_body,
        out_shape=[
            jax.ShapeDtypeStruct((n_pad, d), jnp.float32),
            jax.ShapeDtypeStruct((n_pad, 1), jnp.float32),
        ],
    )(x_pad, W1, degp.T)

    acc1 = layer_kernel(y1[:n], srcp4, dstp4, ewp4, cnt)

    y2 = pl.pallas_call(
        _tc2_body,
        out_shape=jax.ShapeDtypeStruct((n_pad, o), jnp.float32),
    )(acc1, y1, dinv, b1.reshape(1, -1), W2)

    acc2 = layer_kernel(y2[:n], srcp4, dstp4, ewp4, cnt)

    out = pl.pallas_call(
        _tc3_body,
        out_shape=jax.ShapeDtypeStruct((n_pad, o), jnp.float32),
    )(acc2, y2, dinv, b2.reshape(1, -1))

    return out[:n]


# speedup vs baseline: 1.5053x; 1.3100x over previous
import functools

import jax
import jax.numpy as jnp
from jax import lax
from jax.experimental import pallas as pl
from jax.experimental.pallas import tpu as pltpu
from jax.experimental.pallas import tpu_sc as plsc

NC = 2
NS = 16
NW = NC * NS
CHUNK = 128
GCH = 16
SB = 4

_mesh = plsc.VectorSubcoreMesh(core_axis_name="c", subcore_axis_name="s")


def _deg_body(n_pad, k, dst_hbm, ew_hbm, degp_hbm, dstv, ewv, deg_sh):
    cid = lax.axis_index("c")
    sid = lax.axis_index("s")
    wid = cid * NS + sid
    rows_per_sub = n_pad // NS

    for i in range(CHUNK // 16):
        ewv[0, pl.ds(i * 16, 16)] = jnp.zeros((16,), jnp.float32)

    @pl.loop(0, rows_per_sub // CHUNK)
    def _zcopy(i):
        pltpu.sync_copy(
            ewv.at[0],
            deg_sh.at[pl.ds(sid * rows_per_sub + i * CHUNK, CHUNK)],
        )

    plsc.subcore_barrier()

    pltpu.sync_copy(dst_hbm.at[wid], dstv)
    pltpu.sync_copy(ew_hbm.at[wid], ewv)

    @pl.loop(0, k)
    def _scatter(j):
        pltpu.sync_copy(ewv.at[j], deg_sh.at[dstv.at[j]], add=True)

    plsc.subcore_barrier()

    pltpu.sync_copy(
        deg_sh.at[pl.ds(sid * rows_per_sub, rows_per_sub)],
        degp_hbm.at[cid, pl.ds(sid * rows_per_sub, rows_per_sub)],
    )


def _make_deg_kernel(n_pad, k):
    return pl.kernel(
        functools.partial(_deg_body, n_pad, k),
        out_type=jax.ShapeDtypeStruct((NC, n_pad), jnp.float32),
        mesh=_mesh,
        scratch_types=[
            pltpu.VMEM((k, CHUNK), jnp.int32),
            pltpu.VMEM((k, CHUNK), jnp.float32),
            pltpu.VMEM_SHARED((n_pad,), jnp.float32),
        ],
    )


def _part_body(per_w, half, src_hbm, dst_hbm, ew_hbm,
               srcp_hbm, dstp_hbm, ewp_hbm, cnt_hbm,
               in_s, in_d, in_w, st_s, st_d, st_w, st_p, zq,
               cntv, sh_s, sh_d, sh_w):
    cid = lax.axis_index("c")
    sid = lax.axis_index("s")
    wid = cid * NS + sid
    base = sid * per_w
    S0 = NS * per_w

    for i in range(CHUNK // 16):
        st_s[pl.ds(i * 16, 16)] = jnp.zeros((16,), jnp.int32)
        st_w[pl.ds(i * 16, 16)] = jnp.zeros((16,), jnp.float32)
    for r in range(4):
        for i in range(2):
            zq[r, pl.ds(i * 16, 16)] = jnp.zeros((16,), jnp.int32)

    @pl.loop(0, per_w // CHUNK)
    def _z(i):
        for reg in range(2):
            off = reg * S0 + base + i * CHUNK
            pltpu.sync_copy(st_s, sh_s.at[pl.ds(off, CHUNK)])
            pltpu.sync_copy(st_s, sh_d.at[pl.ds(off, CHUNK)])
            pltpu.sync_copy(st_w, sh_w.at[pl.ds(off, CHUNK)])

    pltpu.sync_copy(src_hbm.at[wid], in_s)
    pltpu.sync_copy(dst_hbm.at[wid], in_d)
    pltpu.sync_copy(ew_hbm.at[wid], in_w)

    iota16 = lax.iota(jnp.int32, 16)

    @pl.loop(0, per_w // CHUNK, init_carry=(jnp.int32(0), jnp.int32(0)))
    def _compact(ch, carry):
        off0c, off1c = carry

        @pl.loop(0, CHUNK // 16, init_carry=(off0c, off1c))
        def _grp(g, c2):
            off0, off1 = c2
            s16 = in_s[pl.ds(ch * CHUNK + g * 16, 16)]
            d16 = in_d[pl.ds(ch * CHUNK + g * 16, 16)]
            w16 = in_w[pl.ds(ch * CHUNK + g * 16, 16)]
            m = jnp.clip(half - d16, 0, 1)
            v = m
            for stp, k in enumerate((1, 2, 4, 8)):
                zq[stp, pl.ds(k, 16)] = v
                v = v + zq[stp, pl.ds(0, 16)]
            excl = v - m
            tot0 = v[15]
            mn = 1 - m
            pos = (m * ((base + off0) + excl)
                   + mn * ((S0 + base + off1) + (iota16 - excl)))
            st_s[pl.ds(g * 16, 16)] = s16
            st_d[pl.ds(g * 16, 16)] = d16 - mn * half
            st_w[pl.ds(g * 16, 16)] = w16
            st_p[0, pl.ds(g * 16, 16)] = pos
            return (off0 + tot0, off1 + (16 - tot0))

        off0n, off1n = _grp
        pltpu.sync_copy(st_s, sh_s.at[st_p.at[0]], add=True)
        pltpu.sync_copy(st_d, sh_d.at[st_p.at[0]], add=True)
        pltpu.sync_copy(st_w, sh_w.at[st_p.at[0]], add=True)
        return (off0n, off1n)

    off0, off1 = _compact

    def padded(c):
        g2 = 2 * GCH
        sh = g2.bit_length() - 1
        return jnp.maximum(((c + g2 - 1) >> sh) << sh, g2)

    oh0 = jnp.clip(1 - iota16 * iota16, 0, 1)
    cntv[0, pl.ds(0, 16)] = oh0 * padded(off0)
    cntv[1, pl.ds(0, 16)] = oh0 * padded(off1)

    pltpu.sync_copy(cntv.at[0], cnt_hbm.at[0, wid])
    pltpu.sync_copy(cntv.at[1], cnt_hbm.at[1, wid])
    pltpu.sync_copy(sh_s.at[pl.ds(base, per_w)], srcp_hbm.at[0, wid])
    pltpu.sync_copy(sh_d.at[pl.ds(base, per_w)], dstp_hbm.at[0, wid])
    pltpu.sync_copy(sh_w.at[pl.ds(base, per_w)], ewp_hbm.at[0, wid])
    pltpu.sync_copy(sh_s.at[pl.ds(S0 + base, per_w)], srcp_hbm.at[1, wid])
    pltpu.sync_copy(sh_d.at[pl.ds(S0 + base, per_w)], dstp_hbm.at[1, wid])
    pltpu.sync_copy(sh_w.at[pl.ds(S0 + base, per_w)], ewp_hbm.at[1, wid])


def _make_part_kernel(per_w, half):
    return pl.kernel(
        functools.partial(_part_body, per_w, half),
        out_type=[
            jax.ShapeDtypeStruct((2, NW, per_w), jnp.int32),
            jax.ShapeDtypeStruct((2, NW, per_w), jnp.int32),
            jax.ShapeDtypeStruct((2, NW, per_w), jnp.float32),
            jax.ShapeDtypeStruct((2, NW, 16), jnp.int32),
        ],
        mesh=_mesh,
        scratch_types=[
            pltpu.VMEM((per_w,), jnp.int32),
            pltpu.VMEM((per_w,), jnp.int32),
            pltpu.VMEM((per_w,), jnp.float32),
            pltpu.VMEM((CHUNK,), jnp.int32),
            pltpu.VMEM((CHUNK,), jnp.int32),
            pltpu.VMEM((CHUNK,), jnp.float32),
            pltpu.VMEM((1, CHUNK), jnp.int32),
            pltpu.VMEM((4, 32), jnp.int32),
            pltpu.VMEM((2, 16), jnp.int32),
            pltpu.VMEM_SHARED((2 * NS * per_w,), jnp.int32),
            pltpu.VMEM_SHARED((2 * NS * per_w,), jnp.int32),
            pltpu.VMEM_SHARED((2 * NS * per_w,), jnp.float32),
        ],
    )


def _layer_body(n, n2, half, kmax, d, y_hbm, srcp_hbm, dstp_hbm, ewp_hbm,
                cnt_hbm, acc_hbm, srcv, dstv, ewv, rows, cntv, y_sh, acc_sh,
                gsem, ssem):
    cid = lax.axis_index("c")
    sid = lax.axis_index("s")
    nq = d // 16
    acc_rows_per_sub = half // NS
    y_full = n2 // NS
    y_last = n - (NS - 1) * y_full

    @pl.loop(0, GCH)
    def _zr(r):
        for q in range(nq):
            rows[0, r, pl.ds(q * 16, 16)] = jnp.zeros((16,), jnp.float32)

    @pl.loop(0, acc_rows_per_sub // GCH)
    def _zcopy(i):
        pltpu.sync_copy(
            rows.at[0],
            acc_sh.at[pl.ds(sid * acc_rows_per_sub + i * GCH, GCH)],
        )

    @pl.when(sid < NS - 1)
    def _ycopy_full():
        pltpu.sync_copy(
            y_hbm.at[pl.ds(sid * y_full, y_full)],
            y_sh.at[pl.ds(sid * y_full, y_full)],
        )

    @pl.when(sid == NS - 1)
    def _ycopy_last():
        pltpu.sync_copy(
            y_hbm.at[pl.ds((NS - 1) * y_full, y_last)],
            y_sh.at[pl.ds((NS - 1) * y_full, y_last)],
        )

    plsc.subcore_barrier()

    for seg in range(2):
        w = sid * 2 + seg

        pltpu.sync_copy(cnt_hbm.at[cid, w], cntv.at[0])
        kc = lax.shift_right_logical(cntv[0, pl.ds(0, 16)][0],
                                     GCH.bit_length() - 1)
        kc = jnp.clip(kc, 2, kmax)

        def stage_start(bi, sl):
            pltpu.async_copy(srcp_hbm.at[cid, w, pl.ds(bi * SB, SB)], srcv.at[sl], ssem)
            pltpu.async_copy(dstp_hbm.at[cid, w, pl.ds(bi * SB, SB)], dstv.at[sl], ssem)
            pltpu.async_copy(ewp_hbm.at[cid, w, pl.ds(bi * SB, SB)], ewv.at[sl], ssem)

        def stage_wait(bi, sl):
            pltpu.make_async_copy(srcp_hbm.at[cid, w, pl.ds(bi * SB, SB)], srcv.at[sl], ssem).wait()
            pltpu.make_async_copy(dstp_hbm.at[cid, w, pl.ds(bi * SB, SB)], dstv.at[sl], ssem).wait()
            pltpu.make_async_copy(ewp_hbm.at[cid, w, pl.ds(bi * SB, SB)], ewv.at[sl], ssem).wait()

        def start_gather(j, b):
            sl = (j // SB) % 2
            pltpu.async_copy(y_sh.at[srcv.at[sl, j % SB]], rows.at[b], gsem)

        def wait_gather(j, b):
            sl = (j // SB) % 2
            pltpu.make_async_copy(y_sh.at[srcv.at[sl, j % SB]], rows.at[b], gsem).wait()

        nb = lax.shift_right_logical(kc + SB - 1, SB.bit_length() - 1)

        stage_start(0, 0)
        stage_wait(0, 0)

        @pl.when(nb > 1)
        def _s1():
            stage_start(1, 1)

        start_gather(0, 0)
        start_gather(1, 1)

        @pl.loop(0, lax.shift_right_logical(kc, 1))
        def _outer(j0):
            for b in range(2):
                j = j0 * 2 + b
                bi = j // SB
                jj = j % SB
                sl = bi % 2
                wait_gather(j, b)

                @pl.loop(0, GCH // 16)
                def _scale(g):
                    ws = ewv[sl, jj, pl.ds(g * 16, 16)]
                    for t in range(16):
                        s = ws[t]
                        for q in range(nq):
                            rows[b, g * 16 + t, pl.ds(q * 16, 16)] = (
                                rows[b, g * 16 + t, pl.ds(q * 16, 16)] * s)

                pltpu.sync_copy(rows.at[b], acc_sh.at[dstv.at[sl, jj]], add=True)

                @pl.when(jnp.logical_and(j + 2 < kc, jj == SB - 2))
                def _wstage():
                    stage_wait(bi + 1, 1 - sl)

                @pl.when(j + 2 < kc)
                def _next():
                    start_gather(j + 2, b)

                @pl.when(jnp.logical_and(jj == SB - 1, bi + 2 < nb))
                def _rstage():
                    stage_start(bi + 2, sl)

    plsc.subcore_barrier()

    pltpu.sync_copy(
        acc_sh.at[pl.ds(sid * acc_rows_per_sub, acc_rows_per_sub)],
        acc_hbm.at[pl.ds(cid * half + sid * acc_rows_per_sub,
                         acc_rows_per_sub)],
    )


def _make_layer_kernel(n, n2, half, kmax, d):
    return pl.kernel(
        functools.partial(_layer_body, n, n2, half, kmax, d),
        out_type=jax.ShapeDtypeStruct((2 * half, d), jnp.float32),
        mesh=_mesh,
        scratch_types=[
            pltpu.VMEM((2, SB, GCH), jnp.int32),
            pltpu.VMEM((2, SB, GCH), jnp.int32),
            pltpu.VMEM((2, SB, GCH), jnp.float32),
            pltpu.VMEM((2, GCH, d), jnp.float32),
            pltpu.VMEM((1, 16), jnp.int32),
            pltpu.VMEM_SHARED((n, d), jnp.float32),
            pltpu.VMEM_SHARED((half, d), jnp.float32),
            pltpu.SemaphoreType.DMA,
            pltpu.SemaphoreType.DMA,
        ],
    )


def _tc1_body(x_ref, w_ref, degp_ref, y_ref, dinv_ref):
    deg = degp_ref[:, 0:1] + degp_ref[:, 1:2] + 1.0
    dinv = lax.rsqrt(deg)
    xw = jnp.dot(x_ref[...], w_ref[...], preferred_element_type=jnp.float32)
    y_ref[...] = xw * dinv
    dinv_ref[...] = dinv


def _tc2_body(acc_ref, y_ref, dinv_ref, b_ref, w_ref, y2_ref):
    acc = acc_ref[...] + y_ref[...]
    h = jnp.maximum(dinv_ref[...] * acc + b_ref[...], 0.0)
    y2_ref[...] = jnp.dot(h, w_ref[...], preferred_element_type=jnp.float32) * dinv_ref[...]


def _tc3_body(acc_ref, y_ref, dinv_ref, b_ref, out_ref):
    acc = acc_ref[...] + y_ref[...]
    out_ref[...] = dinv_ref[...] * acc + b_ref[...]


def kernel(x, edge_index, edge_weight, W1, b1, W2, b2):
    n, d = x.shape
    e = edge_weight.shape[0]
    o = W2.shape[1]

    n_pad = ((n + NS * CHUNK - 1) // (NS * CHUNK)) * (NS * CHUNK)
    half = n_pad // 2
    n2 = ((n + NS * 8 - 1) // (NS * 8)) * (NS * 8)
    step = NW * 128
    e_pad = ((e + step - 1) // step) * step
    per_w = e_pad // NW
    kd = per_w // CHUNK
    kmax = per_w // GCH

    src = jnp.concatenate([edge_index[0], jnp.zeros((e_pad - e,), jnp.int32)])
    dst = jnp.concatenate([edge_index[1], jnp.zeros((e_pad - e,), jnp.int32)])
    ew = jnp.concatenate([edge_weight, jnp.zeros((e_pad - e,), jnp.float32)])
    src_w = src.reshape(NW, per_w)
    dst_w = dst.reshape(NW, per_w)
    ew_w = ew.reshape(NW, per_w)
    dst_wd = dst.reshape(NW, kd, CHUNK)
    ew_wd = ew.reshape(NW, kd, CHUNK)
    x_pad = jnp.concatenate([x, jnp.zeros((n_pad - n, d), x.dtype)], axis=0)

    deg_kernel = _make_deg_kernel(n_pad, kd)
    part_kernel = _make_part_kernel(per_w, half)
    layer_kernel = _make_layer_kernel(n, n2, half, kmax, d)

    degp = deg_kernel(dst_wd, ew_wd)
    srcp, dstp, ewp, cnt = part_kernel(src_w, dst_w, ew_w)
    srcp4 = srcp.reshape(2, NW, kmax, GCH)
    dstp4 = dstp.reshape(2, NW, kmax, GCH)
    ewp4 = ewp.reshape(2, NW, kmax, GCH)

    y1, dinv = pl.pallas_call(
        _tc1_body,
        out_shape=[
            jax.ShapeDtypeStruct((n_pad, d), jnp.float32),
            jax.ShapeDtypeStruct((n_pad, 1), jnp.float32),
        ],
    )(x_pad, W1, degp.T)

    acc1 = layer_kernel(y1[:n], srcp4, dstp4, ewp4, cnt)

    y2 = pl.pallas_call(
        _tc2_body,
        out_shape=jax.ShapeDtypeStruct((n_pad, o), jnp.float32),
    )(acc1, y1, dinv, b1.reshape(1, -1), W2)

    acc2 = layer_kernel(y2[:n], srcp4, dstp4, ewp4, cnt)

    out = pl.pallas_call(
        _tc3_body,
        out_shape=jax.ShapeDtypeStruct((n_pad, o), jnp.float32),
    )(acc2, y2, dinv, b2.reshape(1, -1))

    return out[:n]
